# baseline (speedup 1.0000x reference)
import functools

import jax
import jax.numpy as jnp
from jax import lax
from jax.experimental import pallas as pl
from jax.experimental.pallas import tpu as pltpu
from jax.experimental.pallas import tpu_sc as plsc

N = 10000
E = 320000
NFEAT = 128
NHID = 64
NHEADS = 4
NCLASS = 40
ALPHA = 0.2

NPAD = 10240
F1 = 80
F2 = 48
K = 80
NB = 125
G = K // 16
CH = NPAD // 16
BN = 1000
EPAD = E + 2 * K
_HIGH = jax.lax.Precision.HIGHEST


def _leaky(z):
    return jnp.where(z > 0, z, ALPHA * z)


def _elu(z):
    return jnp.where(z > 0, z, jnp.exp(z) - 1.0)


def _tc_a_body(x_ref, w0_ref, a0_ref, htab_ref, eab_ref):
    xb = x_ref[...]
    rows = []
    als = []
    bes = []
    ones = jnp.ones((BN, 1), jnp.float32)
    zpad = jnp.zeros((BN, F1 - NHID - 1), jnp.float32)
    for h in range(NHEADS):
        hp = jnp.dot(xb, w0_ref[h], preferred_element_type=jnp.float32,
                     precision=_HIGH)
        als.append(jnp.dot(hp, a0_ref[h, :NHID].reshape(NHID, 1),
                           preferred_element_type=jnp.float32, precision=_HIGH))
        bes.append(jnp.dot(hp, a0_ref[h, NHID:].reshape(NHID, 1),
                           preferred_element_type=jnp.float32, precision=_HIGH))
        rows.append(jnp.concatenate([hp, ones, zpad], axis=1))
    htab_ref[...] = jnp.stack(rows, axis=0)
    eab_ref[...] = jnp.concatenate(
        als + bes + [jnp.zeros((BN, 8), jnp.float32)], axis=1)


def _tc_a(x, w0, a0):
    return pl.pallas_call(
        _tc_a_body,
        grid=(N // BN,),
        in_specs=[
            pl.BlockSpec((BN, NFEAT), lambda i: (i, 0)),
            pl.BlockSpec((NHEADS, NFEAT, NHID), lambda i: (0, 0, 0)),
            pl.BlockSpec((NHEADS, 2 * NHID), lambda i: (0, 0)),
        ],
        out_specs=[
            pl.BlockSpec((NHEADS, BN, F1), lambda i: (0, i, 0)),
            pl.BlockSpec((BN, 16), lambda i: (i, 0)),
        ],
        out_shape=[
            jax.ShapeDtypeStruct((NHEADS, N, F1), jnp.float32),
            jax.ShapeDtypeStruct((N, 16), jnp.float32),
        ],
    )(x, w0, a0)


def _make_sc_gat(nheads, F):
    mesh = plsc.VectorSubcoreMesh(core_axis_name="c", subcore_axis_name="s")
    fq = F // 16

    @functools.partial(
        pl.kernel,
        out_type=jax.ShapeDtypeStruct((nheads * 2 * NPAD, F), jnp.float32),
        mesh=mesh,
        compiler_params=pltpu.CompilerParams(
            needs_layout_passes=False, use_tc_tiling_on_sc=False),
        scratch_types=[
            pltpu.VMEM((NPAD,), jnp.float32),
            pltpu.VMEM((NPAD,), jnp.float32),
            pltpu.VMEM((NPAD,), jnp.float32),
            pltpu.VMEM((K,), jnp.int32),
            *[pltpu.VMEM((K,), jnp.int32) for _ in range(3)],
            *[pltpu.VMEM((K,), jnp.int32) for _ in range(3)],
            *[pltpu.VMEM((K,), jnp.float32) for _ in range(3)],
            *[pltpu.VMEM((K, F), jnp.float32) for _ in range(3)],
            pltpu.VMEM_SHARED((NPAD, F), jnp.float32),
            *[pltpu.SemaphoreType.DMA for _ in range(6)],
        ],
    )
    def sc(htab, srcE, dstE, abm, zrows, acc_out,
           a_b, b_b, m_b, s_b, d0, d1, d2, g0, g1, g2, p0, p1, p2,
           r0, r1, r2, accsp, gs0, gs1, gs2, ss0, ss1, ss2):
        c = lax.axis_index("c")
        t = lax.axis_index("s")
        D = (d0, d1, d2)
        Gb = (g0, g1, g2)
        P = (p0, p1, p2)
        R = (r0, r1, r2)
        GS = (gs0, gs1, gs2)
        SS = (ss0, ss1, ss2)

        for h in range(nheads):
            pltpu.sync_copy(abm.at[3 * h + 0], a_b)
            pltpu.sync_copy(abm.at[3 * h + 1], b_b)
            pltpu.sync_copy(abm.at[3 * h + 2], m_b)
            pltpu.sync_copy(zrows, accsp.at[pl.ds(t * CH, CH)])
            plsc.subcore_barrier()

            ebase = (c * 16 + t) * (K * NB)

            def pre(b, i):
                base = ebase + b * K
                pltpu.sync_copy(srcE.at[pl.ds(base, K)], s_b)
                pltpu.sync_copy(dstE.at[pl.ds(base, K)], D[i])

                def grp(g, _):
                    o = g * 16
                    sv = s_b[pl.ds(o, 16)]
                    dv = D[i][pl.ds(o, 16)]
                    av = plsc.load_gather(a_b, [dv])
                    bv = plsc.load_gather(b_b, [sv])
                    mv = plsc.load_gather(m_b, [dv])
                    P[i][pl.ds(o, 16)] = jnp.exp(_leaky(av + bv) - mv)
                    Gb[i][pl.ds(o, 16)] = sv + (h * N)
                    return 0
                lax.fori_loop(0, G, grp, 0)
                pltpu.async_copy(htab.at[Gb[i]], R[i], GS[i])

            def fire(i):
                pltpu.make_async_copy(htab.at[Gb[i]], R[i], GS[i]).wait()

                def sg(g, _):
                    o = g * 16
                    pv16 = P[i][pl.ds(o, 16)]
                    for lane in range(16):
                        idx = jnp.full((16,), lane, jnp.int32)
                        pv = pv16.at[idx].get(mode="promise_in_bounds")
                        r = o + lane
                        for q in range(fq):
                            R[i][r, pl.ds(q * 16, 16)] = (
                                R[i][r, pl.ds(q * 16, 16)] * pv)
                    return 0
                lax.fori_loop(0, G, sg, 0)
                pltpu.async_copy(R[i], accsp.at[D[i]], SS[i], add=True)

            def drain_sc(i):
                pltpu.make_async_copy(R[i], accsp.at[D[i]], SS[i]).wait()

            pre(0, 0)
            pre(1, 1)
            fire(0)
            pre(2, 2)
            fire(1)

            def body(bp, _):
                b0 = 2 + bp * 3
                for j, (ci, ni) in enumerate(((2, 0), (0, 1), (1, 2))):
                    drain_sc(ni)
                    pre(b0 + j + 1, ni)
                    fire(ci)
                return 0
            lax.fori_loop(0, (NB - 2) // 3, body, 0)

            pltpu.make_async_copy(htab.at[Gb[2]], R[2], GS[2]).wait()
            drain_sc(0)
            drain_sc(1)
            plsc.subcore_barrier()

            out_base = (h * 2 + c) * NPAD + t * CH
            pltpu.sync_copy(accsp.at[pl.ds(t * CH, CH)],
                            acc_out.at[pl.ds(out_base, CH)])

    return sc


_sc_l1 = _make_sc_gat(NHEADS, F1)
_sc_l2 = _make_sc_gat(1, F2)


def _tc_b_body(acc_ref, w1_ref, a1_ref, htab2_ref, eab2_ref):
    xs = []
    for h in range(NHEADS):
        d = acc_ref[h, 0] + acc_ref[h, 1]
        o = d[:, :NHID] / (d[:, NHID:NHID + 1] + 1e-16)
        xs.append(_elu(o))
    xcat = jnp.concatenate(xs, axis=1)
    hout = jnp.dot(xcat, w1_ref[...], preferred_element_type=jnp.float32,
                   precision=_HIGH)
    a1v = a1_ref[...]
    al2 = jnp.dot(hout, a1v[:NCLASS].reshape(NCLASS, 1),
                  preferred_element_type=jnp.float32, precision=_HIGH)
    be2 = jnp.dot(hout, a1v[NCLASS:].reshape(NCLASS, 1),
                  preferred_element_type=jnp.float32, precision=_HIGH)
    htab2_ref[...] = jnp.concatenate(
        [hout, jnp.ones((BN, 1), jnp.float32),
         jnp.zeros((BN, F2 - NCLASS - 1), jnp.float32)], axis=1)
    eab2_ref[...] = jnp.concatenate(
        [al2, be2, jnp.zeros((BN, 14), jnp.float32)], axis=1)


def _tc_b(acc1, w1, a1):
    return pl.pallas_call(
        _tc_b_body,
        grid=(N // BN,),
        in_specs=[
            pl.BlockSpec((NHEADS, 2, BN, F1), lambda i: (0, 0, i, 0)),
            pl.BlockSpec((NHEADS * NHID, NCLASS), lambda i: (0, 0)),
            pl.BlockSpec((2 * NCLASS,), lambda i: (0,)),
        ],
        out_specs=[
            pl.BlockSpec((BN, F2), lambda i: (i, 0)),
            pl.BlockSpec((BN, 16), lambda i: (i, 0)),
        ],
        out_shape=[
            jax.ShapeDtypeStruct((N, F2), jnp.float32),
            jax.ShapeDtypeStruct((N, 16), jnp.float32),
        ],
    )(acc1, w1, a1)


def _tc_c_body(acc_ref, out_ref):
    d = acc_ref[0] + acc_ref[1]
    o = d[:, :NCLASS] / (d[:, NCLASS:NCLASS + 1] + 1e-16)
    logits = _elu(o)
    mx = jnp.max(logits, axis=1, keepdims=True)
    ls = logits - mx
    out_ref[...] = ls - jnp.log(jnp.sum(jnp.exp(ls), axis=1, keepdims=True))


def _tc_c(acc2):
    return pl.pallas_call(
        _tc_c_body,
        grid=(N // BN,),
        in_specs=[pl.BlockSpec((2, BN, F2), lambda i: (0, i, 0))],
        out_specs=pl.BlockSpec((BN, NCLASS), lambda i: (i, 0)),
        out_shape=jax.ShapeDtypeStruct((N, NCLASS), jnp.float32),
    )(acc2)


def kernel(x, edge_index, W0, a0, W1, a1):
    src = jnp.pad(edge_index[0], (0, EPAD - E))
    dst = jnp.pad(edge_index[1], (0, EPAD - E))

    htab1, eab = _tc_a(x, W0, a0)
    al = eab[:, :NHEADS]
    be = eab[:, NHEADS:2 * NHEADS]
    m = _leaky(al + jnp.max(be, axis=0, keepdims=True))
    abm1 = jnp.stack([al.T, be.T, m.T], axis=1).reshape(3 * NHEADS, N)
    abm1 = jnp.pad(abm1, ((0, 0), (0, NPAD - N)))

    acc1 = _sc_l1(htab1.reshape(NHEADS * N, F1), src, dst, abm1,
                  jnp.zeros((CH, F1), jnp.float32))
    acc1 = acc1.reshape(NHEADS, 2, NPAD, F1)

    htab2, eab2 = _tc_b(acc1, W1, a1)
    al2 = eab2[:, 0]
    be2 = eab2[:, 1]
    m2 = _leaky(al2 + jnp.max(be2))
    abm2 = jnp.pad(jnp.stack([al2, be2, m2], axis=0), ((0, 0), (0, NPAD - N)))

    acc2 = _sc_l2(htab2, src, dst, abm2, jnp.zeros((CH, F2), jnp.float32))
    return _tc_c(acc2.reshape(2, NPAD, F2))


# speedup vs baseline: 1.6681x; 1.6681x over previous
import functools

import jax
import jax.numpy as jnp
from jax import lax
from jax.experimental import pallas as pl
from jax.experimental.pallas import tpu as pltpu
from jax.experimental.pallas import tpu_sc as plsc

N = 10000
E = 320000
NFEAT = 128
NHID = 64
NHEADS = 4
NCLASS = 40
ALPHA = 0.2

NPAD = 10240
F1 = 80
F2 = 48
K = 80
NB = 125
G = K // 16
CH = NPAD // 16
BN = 1000
EPAD = E + 2 * K
_HIGH = jax.lax.Precision.HIGHEST


def _leaky(z):
    return jnp.where(z > 0, z, ALPHA * z)


def _elu(z):
    return jnp.where(z > 0, z, jnp.exp(z) - 1.0)


def _tc_a_body(x_ref, w0c_ref, am_ref, htab_ref, eab_ref):
    xb = x_ref[...]
    hcat = jnp.dot(xb, w0c_ref[...], preferred_element_type=jnp.float32,
                   precision=_HIGH)
    eab_ref[...] = jnp.dot(hcat, am_ref[...],
                           preferred_element_type=jnp.float32,
                           precision=_HIGH)
    ones = jnp.ones((BN, 1), jnp.float32)
    zpad = jnp.zeros((BN, F1 - NHID - 1), jnp.float32)
    rows = [jnp.concatenate([hcat[:, h * NHID:(h + 1) * NHID], ones, zpad],
                            axis=1) for h in range(NHEADS)]
    htab_ref[...] = jnp.stack(rows, axis=0)


def _tc_a(x, w0cat, amat):
    return pl.pallas_call(
        _tc_a_body,
        grid=(N // BN,),
        in_specs=[
            pl.BlockSpec((BN, NFEAT), lambda i: (i, 0)),
            pl.BlockSpec((NFEAT, NHEADS * NHID), lambda i: (0, 0)),
            pl.BlockSpec((NHEADS * NHID, 16), lambda i: (0, 0)),
        ],
        out_specs=[
            pl.BlockSpec((NHEADS, BN, F1), lambda i: (0, i, 0)),
            pl.BlockSpec((BN, 16), lambda i: (i, 0)),
        ],
        out_shape=[
            jax.ShapeDtypeStruct((NHEADS, N, F1), jnp.float32),
            jax.ShapeDtypeStruct((N, 16), jnp.float32),
        ],
    )(x, w0cat, amat)


def _make_sc_gat(nheads, F):
    mesh = plsc.VectorSubcoreMesh(core_axis_name="c", subcore_axis_name="s")
    fq = F // 16

    @functools.partial(
        pl.kernel,
        out_type=jax.ShapeDtypeStruct((nheads, 2, NPAD, F), jnp.float32),
        mesh=mesh,
        compiler_params=pltpu.CompilerParams(
            needs_layout_passes=False, use_tc_tiling_on_sc=False),
        scratch_types=[
            pltpu.VMEM((NPAD,), jnp.float32),
            pltpu.VMEM((NPAD,), jnp.float32),
            pltpu.VMEM((NPAD,), jnp.float32),
            pltpu.VMEM((K * (NB + 1),), jnp.int32),
            pltpu.VMEM((K * (NB + 1),), jnp.int32),
            *[pltpu.VMEM((K,), jnp.int32) for _ in range(3)],
            *[pltpu.VMEM((K,), jnp.int32) for _ in range(3)],
            *[pltpu.VMEM((K,), jnp.float32) for _ in range(3)],
            *[pltpu.VMEM((K, F), jnp.float32) for _ in range(3)],
            pltpu.VMEM_SHARED((NPAD, F), jnp.float32),
            *[pltpu.SemaphoreType.DMA for _ in range(6)],
        ],
    )
    def sc(htab, srcE, dstE, abm, zrows, acc_out,
           a_b, b_b, m_b, sidx, didx, d0, d1, d2, g0, g1, g2, p0, p1, p2,
           r0, r1, r2, accsp, gs0, gs1, gs2, ss0, ss1, ss2):
        c = lax.axis_index("c")
        t = lax.axis_index("s")
        D = (d0, d1, d2)
        Gb = (g0, g1, g2)
        P = (p0, p1, p2)
        R = (r0, r1, r2)
        GS = (gs0, gs1, gs2)
        SS = (ss0, ss1, ss2)

        ebase0 = (c * 16 + t) * (K * NB)
        pltpu.sync_copy(srcE.at[pl.ds(ebase0, K * (NB + 1))], sidx)
        pltpu.sync_copy(dstE.at[pl.ds(ebase0, K * (NB + 1))], didx)

        for h in range(nheads):
            pltpu.sync_copy(abm.at[3 * h + 0], a_b)
            pltpu.sync_copy(abm.at[3 * h + 1], b_b)
            pltpu.sync_copy(abm.at[3 * h + 2], m_b)
            pltpu.sync_copy(zrows, accsp.at[pl.ds(t * CH, CH)])
            plsc.subcore_barrier()

            def pre(b, i):
                def grp(g, _):
                    o = b * K + g * 16
                    sv = sidx[pl.ds(o, 16)]
                    dv = didx[pl.ds(o, 16)]
                    av = plsc.load_gather(a_b, [dv])
                    bv = plsc.load_gather(b_b, [sv])
                    mv = plsc.load_gather(m_b, [dv])
                    P[i][pl.ds(g * 16, 16)] = jnp.exp(_leaky(av + bv) - mv)
                    Gb[i][pl.ds(g * 16, 16)] = sv
                    D[i][pl.ds(g * 16, 16)] = dv
                    return 0
                lax.fori_loop(0, G, grp, 0)
                pltpu.async_copy(htab.at[h].at[Gb[i]], R[i], GS[i])

            def fire(i):
                pltpu.make_async_copy(htab.at[h].at[Gb[i]], R[i], GS[i]).wait()

                def sg(g, _):
                    o = g * 16
                    pv16 = P[i][pl.ds(o, 16)]
                    for lane in range(16):
                        idx = jnp.full((16,), lane, jnp.int32)
                        pv = pv16.at[idx].get(mode="promise_in_bounds")
                        r = o + lane
                        for q in range(fq):
                            R[i][r, pl.ds(q * 16, 16)] = (
                                R[i][r, pl.ds(q * 16, 16)] * pv)
                    return 0
                lax.fori_loop(0, G, sg, 0)
                pltpu.async_copy(R[i], accsp.at[D[i]], SS[i], add=True)

            def drain_sc(i):
                pltpu.make_async_copy(R[i], accsp.at[D[i]], SS[i]).wait()

            pre(0, 0)
            pre(1, 1)
            fire(0)
            pre(2, 2)
            fire(1)

            def body(bp, _):
                b0 = 2 + bp * 3
                for j, (ci, ni) in enumerate(((2, 0), (0, 1), (1, 2))):
                    drain_sc(ni)
                    pre(b0 + j + 1, ni)
                    fire(ci)
                return 0
            lax.fori_loop(0, (NB - 2) // 3, body, 0)

            pltpu.make_async_copy(htab.at[h].at[Gb[2]], R[2], GS[2]).wait()
            drain_sc(0)
            drain_sc(1)
            plsc.subcore_barrier()

            pltpu.sync_copy(accsp.at[pl.ds(t * CH, CH)],
                            acc_out.at[h].at[c].at[pl.ds(t * CH, CH)])

    return sc


_sc_l1 = _make_sc_gat(NHEADS, F1)
_sc_l2 = _make_sc_gat(1, F2)


def _tc_b_body(acc_ref, w1c_ref, htab2_ref, eab2_ref):
    xs = []
    for h in range(NHEADS):
        d = acc_ref[h, 0] + acc_ref[h, 1]
        o = d[:, :NHID] / (d[:, NHID:NHID + 1] + 1e-16)
        xs.append(_elu(o))
    xcat = jnp.concatenate(xs, axis=1)
    y = jnp.dot(xcat, w1c_ref[...], preferred_element_type=jnp.float32,
                precision=_HIGH)
    htab2_ref[...] = jnp.concatenate(
        [y[:, :NCLASS], jnp.ones((BN, 1), jnp.float32),
         jnp.zeros((BN, F2 - NCLASS - 1), jnp.float32)], axis=1)
    eab2_ref[...] = jnp.concatenate(
        [y[:, NCLASS:NCLASS + 2], jnp.zeros((BN, 14), jnp.float32)], axis=1)


def _tc_b(acc1, w1cat):
    return pl.pallas_call(
        _tc_b_body,
        grid=(N // BN,),
        in_specs=[
            pl.BlockSpec((NHEADS, 2, BN, F1), lambda i: (0, 0, i, 0)),
            pl.BlockSpec((NHEADS * NHID, F2), lambda i: (0, 0)),
        ],
        out_specs=[
            pl.BlockSpec((BN, F2), lambda i: (i, 0)),
            pl.BlockSpec((BN, 16), lambda i: (i, 0)),
        ],
        out_shape=[
            jax.ShapeDtypeStruct((N, F2), jnp.float32),
            jax.ShapeDtypeStruct((N, 16), jnp.float32),
        ],
    )(acc1, w1cat)


def _tc_c_body(acc_ref, out_ref):
    d = acc_ref[0, 0] + acc_ref[0, 1]
    o = d[:, :NCLASS] / (d[:, NCLASS:NCLASS + 1] + 1e-16)
    logits = _elu(o)
    mx = jnp.max(logits, axis=1, keepdims=True)
    ls = logits - mx
    out_ref[...] = ls - jnp.log(jnp.sum(jnp.exp(ls), axis=1, keepdims=True))


def _tc_c(acc2):
    return pl.pallas_call(
        _tc_c_body,
        grid=(N // BN,),
        in_specs=[pl.BlockSpec((1, 2, BN, F2), lambda i: (0, 0, i, 0))],
        out_specs=pl.BlockSpec((BN, NCLASS), lambda i: (i, 0)),
        out_shape=jax.ShapeDtypeStruct((N, NCLASS), jnp.float32),
    )(acc2)


def kernel(x, edge_index, W0, a0, W1, a1):
    src = jnp.pad(edge_index[0], (0, EPAD - E))
    dst = jnp.pad(edge_index[1], (0, EPAD - E))

    w0cat = jnp.transpose(W0, (1, 0, 2)).reshape(NFEAT, NHEADS * NHID)
    amat = jnp.zeros((NHEADS, NHID, 16), jnp.float32)
    for h in range(NHEADS):
        amat = amat.at[h, :, h].set(a0[h, :NHID])
        amat = amat.at[h, :, NHEADS + h].set(a0[h, NHID:])
    amat = amat.reshape(NHEADS * NHID, 16)
    w1cat = jnp.concatenate(
        [W1, (W1 @ a1[:NCLASS])[:, None], (W1 @ a1[NCLASS:])[:, None],
         jnp.zeros((NHEADS * NHID, F2 - NCLASS - 2), jnp.float32)], axis=1)

    htab1, eab = _tc_a(x, w0cat, amat)
    al = eab[:, :NHEADS]
    be = eab[:, NHEADS:2 * NHEADS]
    m = _leaky(al + jnp.max(be, axis=0, keepdims=True))
    abm1 = jnp.stack([al.T, be.T, m.T], axis=1).reshape(3 * NHEADS, N)
    abm1 = jnp.pad(abm1, ((0, 0), (0, NPAD - N)))

    acc1 = _sc_l1(htab1, src, dst, abm1, jnp.zeros((CH, F1), jnp.float32))

    htab2, eab2 = _tc_b(acc1, w1cat)
    al2 = eab2[:, 0]
    be2 = eab2[:, 1]
    m2 = _leaky(al2 + jnp.max(be2))
    abm2 = jnp.pad(jnp.stack([al2, be2, m2], axis=0), ((0, 0), (0, NPAD - N)))

    acc2 = _sc_l2(htab2[None], src, dst, abm2,
                  jnp.zeros((CH, F2), jnp.float32))
    return _tc_c(acc2)


# speedup vs baseline: 1.7245x; 1.0338x over previous
import functools

import jax
import jax.numpy as jnp
from jax import lax
from jax.experimental import pallas as pl
from jax.experimental.pallas import tpu as pltpu
from jax.experimental.pallas import tpu_sc as plsc

N = 10000
E = 320000
NFEAT = 128
NHID = 64
NHEADS = 4
NCLASS = 40
ALPHA = 0.2

NPAD = 10240
F1 = 80
F2 = 48
K = 80
NB = 125
G = K // 16
CH = NPAD // 16
BN = 1000
EPAD = E + 2 * K
_HIGH = jax.lax.Precision.DEFAULT


def _leaky(z):
    return jnp.where(z > 0, z, ALPHA * z)


def _elu(z):
    return jnp.where(z > 0, z, jnp.exp(z) - 1.0)


def _tc_a_body(x_ref, w0c_ref, am_ref, htab_ref, eab_ref):
    xb = x_ref[...]
    hcat = jnp.dot(xb, w0c_ref[...], preferred_element_type=jnp.float32,
                   precision=_HIGH)
    eab_ref[...] = jnp.dot(hcat, am_ref[...],
                           preferred_element_type=jnp.float32,
                           precision=_HIGH)
    ones = jnp.ones((BN, 1), jnp.float32)
    zpad = jnp.zeros((BN, F1 - NHID - 1), jnp.float32)
    rows = [jnp.concatenate([hcat[:, h * NHID:(h + 1) * NHID], ones, zpad],
                            axis=1) for h in range(NHEADS)]
    htab_ref[...] = jnp.stack(rows, axis=0)


def _tc_a(x, w0cat, amat):
    return pl.pallas_call(
        _tc_a_body,
        grid=(N // BN,),
        in_specs=[
            pl.BlockSpec((BN, NFEAT), lambda i: (i, 0)),
            pl.BlockSpec((NFEAT, NHEADS * NHID), lambda i: (0, 0)),
            pl.BlockSpec((NHEADS * NHID, 16), lambda i: (0, 0)),
        ],
        out_specs=[
            pl.BlockSpec((NHEADS, BN, F1), lambda i: (0, i, 0)),
            pl.BlockSpec((BN, 16), lambda i: (i, 0)),
        ],
        out_shape=[
            jax.ShapeDtypeStruct((NHEADS, N, F1), jnp.float32),
            jax.ShapeDtypeStruct((N, 16), jnp.float32),
        ],
    )(x, w0cat, amat)


def _make_sc_gat(nheads, F):
    mesh = plsc.VectorSubcoreMesh(core_axis_name="c", subcore_axis_name="s")
    fq = F // 16

    @functools.partial(
        pl.kernel,
        out_type=jax.ShapeDtypeStruct((nheads, 2, NPAD, F), jnp.float32),
        mesh=mesh,
        compiler_params=pltpu.CompilerParams(
            needs_layout_passes=False, use_tc_tiling_on_sc=False),
        scratch_types=[
            pltpu.VMEM((NPAD,), jnp.float32),
            pltpu.VMEM((NPAD,), jnp.float32),
            pltpu.VMEM((NPAD,), jnp.float32),
            pltpu.VMEM((K * (NB + 1),), jnp.int32),
            pltpu.VMEM((K * (NB + 1),), jnp.int32),
            *[pltpu.VMEM((K,), jnp.int32) for _ in range(3)],
            *[pltpu.VMEM((K,), jnp.int32) for _ in range(3)],
            *[pltpu.VMEM((K,), jnp.float32) for _ in range(3)],
            *[pltpu.VMEM((K, F), jnp.float32) for _ in range(3)],
            pltpu.VMEM_SHARED((NPAD, F), jnp.float32),
            *[pltpu.SemaphoreType.DMA for _ in range(6)],
        ],
    )
    def sc(htab, srcE, dstE, abm, zrows, acc_out,
           a_b, b_b, m_b, sidx, didx, d0, d1, d2, g0, g1, g2, p0, p1, p2,
           r0, r1, r2, accsp, gs0, gs1, gs2, ss0, ss1, ss2):
        c = lax.axis_index("c")
        t = lax.axis_index("s")
        D = (d0, d1, d2)
        Gb = (g0, g1, g2)
        P = (p0, p1, p2)
        R = (r0, r1, r2)
        GS = (gs0, gs1, gs2)
        SS = (ss0, ss1, ss2)

        ebase0 = (c * 16 + t) * (K * NB)
        pltpu.sync_copy(srcE.at[pl.ds(ebase0, K * (NB + 1))], sidx)
        pltpu.sync_copy(dstE.at[pl.ds(ebase0, K * (NB + 1))], didx)

        for h in range(nheads):
            pltpu.sync_copy(abm.at[3 * h + 0], a_b)
            pltpu.sync_copy(abm.at[3 * h + 1], b_b)
            pltpu.sync_copy(abm.at[3 * h + 2], m_b)
            pltpu.sync_copy(zrows, accsp.at[pl.ds(t * CH, CH)])
            plsc.subcore_barrier()

            def pre(b, i):
                def grp(g, _):
                    o = b * K + g * 16
                    sv = sidx[pl.ds(o, 16)]
                    dv = didx[pl.ds(o, 16)]
                    av = plsc.load_gather(a_b, [dv])
                    bv = plsc.load_gather(b_b, [sv])
                    mv = plsc.load_gather(m_b, [dv])
                    P[i][pl.ds(g * 16, 16)] = jnp.exp(_leaky(av + bv) - mv)
                    Gb[i][pl.ds(g * 16, 16)] = sv
                    D[i][pl.ds(g * 16, 16)] = dv
                    return 0
                lax.fori_loop(0, G, grp, 0)
                pltpu.async_copy(htab.at[h].at[Gb[i]], R[i], GS[i])

            def fire(i):
                pltpu.make_async_copy(htab.at[h].at[Gb[i]], R[i], GS[i]).wait()

                def sg(g, _):
                    o = g * 16
                    pv16 = P[i][pl.ds(o, 16)]
                    for lane in range(16):
                        idx = jnp.full((16,), lane, jnp.int32)
                        pv = pv16.at[idx].get(mode="promise_in_bounds")
                        r = o + lane
                        for q in range(fq):
                            R[i][r, pl.ds(q * 16, 16)] = (
                                R[i][r, pl.ds(q * 16, 16)] * pv)
                    return 0
                lax.fori_loop(0, G, sg, 0)
                pltpu.async_copy(R[i], accsp.at[D[i]], SS[i], add=True)

            def drain_sc(i):
                pltpu.make_async_copy(R[i], accsp.at[D[i]], SS[i]).wait()

            pre(0, 0)
            pre(1, 1)
            fire(0)
            pre(2, 2)
            fire(1)

            def body(bp, _):
                b0 = 2 + bp * 3
                for j, (ci, ni) in enumerate(((2, 0), (0, 1), (1, 2))):
                    drain_sc(ni)
                    pre(b0 + j + 1, ni)
                    fire(ci)
                return 0
            lax.fori_loop(0, (NB - 2) // 3, body, 0)

            pltpu.make_async_copy(htab.at[h].at[Gb[2]], R[2], GS[2]).wait()
            drain_sc(0)
            drain_sc(1)
            plsc.subcore_barrier()

            pltpu.sync_copy(accsp.at[pl.ds(t * CH, CH)],
                            acc_out.at[h].at[c].at[pl.ds(t * CH, CH)])

    return sc


_sc_l1 = _make_sc_gat(NHEADS, F1)
_sc_l2 = _make_sc_gat(1, F2)


def _tc_b_body(acc_ref, w1c_ref, htab2_ref, eab2_ref):
    xs = []
    for h in range(NHEADS):
        d = acc_ref[h, 0] + acc_ref[h, 1]
        o = d[:, :NHID] / (d[:, NHID:NHID + 1] + 1e-16)
        xs.append(_elu(o))
    xcat = jnp.concatenate(xs, axis=1)
    y = jnp.dot(xcat, w1c_ref[...], preferred_element_type=jnp.float32,
                precision=_HIGH)
    htab2_ref[...] = jnp.concatenate(
        [y[:, :NCLASS], jnp.ones((BN, 1), jnp.float32),
         jnp.zeros((BN, F2 - NCLASS - 1), jnp.float32)], axis=1)
    eab2_ref[...] = jnp.concatenate(
        [y[:, NCLASS:NCLASS + 2], jnp.zeros((BN, 14), jnp.float32)], axis=1)


def _tc_b(acc1, w1cat):
    return pl.pallas_call(
        _tc_b_body,
        grid=(N // BN,),
        in_specs=[
            pl.BlockSpec((NHEADS, 2, BN, F1), lambda i: (0, 0, i, 0)),
            pl.BlockSpec((NHEADS * NHID, F2), lambda i: (0, 0)),
        ],
        out_specs=[
            pl.BlockSpec((BN, F2), lambda i: (i, 0)),
            pl.BlockSpec((BN, 16), lambda i: (i, 0)),
        ],
        out_shape=[
            jax.ShapeDtypeStruct((N, F2), jnp.float32),
            jax.ShapeDtypeStruct((N, 16), jnp.float32),
        ],
    )(acc1, w1cat)


def _tc_c_body(acc_ref, out_ref):
    d = acc_ref[0, 0] + acc_ref[0, 1]
    o = d[:, :NCLASS] / (d[:, NCLASS:NCLASS + 1] + 1e-16)
    logits = _elu(o)
    mx = jnp.max(logits, axis=1, keepdims=True)
    ls = logits - mx
    out_ref[...] = ls - jnp.log(jnp.sum(jnp.exp(ls), axis=1, keepdims=True))


def _tc_c(acc2):
    return pl.pallas_call(
        _tc_c_body,
        grid=(N // BN,),
        in_specs=[pl.BlockSpec((1, 2, BN, F2), lambda i: (0, 0, i, 0))],
        out_specs=pl.BlockSpec((BN, NCLASS), lambda i: (i, 0)),
        out_shape=jax.ShapeDtypeStruct((N, NCLASS), jnp.float32),
    )(acc2)


def kernel(x, edge_index, W0, a0, W1, a1):
    src = jnp.pad(edge_index[0], (0, EPAD - E))
    dst = jnp.pad(edge_index[1], (0, EPAD - E))

    w0cat = jnp.transpose(W0, (1, 0, 2)).reshape(NFEAT, NHEADS * NHID)
    amat = jnp.zeros((NHEADS, NHID, 16), jnp.float32)
    for h in range(NHEADS):
        amat = amat.at[h, :, h].set(a0[h, :NHID])
        amat = amat.at[h, :, NHEADS + h].set(a0[h, NHID:])
    amat = amat.reshape(NHEADS * NHID, 16)
    w1cat = jnp.concatenate(
        [W1, (W1 @ a1[:NCLASS])[:, None], (W1 @ a1[NCLASS:])[:, None],
         jnp.zeros((NHEADS * NHID, F2 - NCLASS - 2), jnp.float32)], axis=1)

    htab1, eab = _tc_a(x, w0cat, amat)
    al = eab[:, :NHEADS]
    be = eab[:, NHEADS:2 * NHEADS]
    m = _leaky(al + jnp.max(be, axis=0, keepdims=True))
    abm1 = jnp.stack([al.T, be.T, m.T], axis=1).reshape(3 * NHEADS, N)
    abm1 = jnp.pad(abm1, ((0, 0), (0, NPAD - N)))

    acc1 = _sc_l1(htab1, src, dst, abm1, jnp.zeros((CH, F1), jnp.float32))

    htab2, eab2 = _tc_b(acc1, w1cat)
    al2 = eab2[:, 0]
    be2 = eab2[:, 1]
    m2 = _leaky(al2 + jnp.max(be2))
    abm2 = jnp.pad(jnp.stack([al2, be2, m2], axis=0), ((0, 0), (0, NPAD - N)))

    acc2 = _sc_l2(htab2[None], src, dst, abm2,
                  jnp.zeros((CH, F2), jnp.float32))
    return _tc_c(acc2)
